# baseline (speedup 1.0000x reference)
import functools

import jax
import jax.numpy as jnp
from jax import lax
from jax.experimental import pallas as pl
from jax.experimental.pallas import tpu as pltpu
from jax.experimental.pallas import tpu_sc as plsc


def kernel(_input, weights):
    bsz, seq_len = _input.shape
    embed_dim = weights.shape[1]

    info = plsc.get_sparse_core_info()
    nc, ns = info.num_cores, info.num_subcores
    nw = nc * ns
    rows_per_w = seq_len // nw

    mesh = plsc.VectorSubcoreMesh(core_axis_name="c", subcore_axis_name="s")

    @functools.partial(
        pl.kernel,
        mesh=mesh,
        out_type=jax.ShapeDtypeStruct((bsz * seq_len, embed_dim), jnp.float32),
        scratch_types=[
            pltpu.SemaphoreType.DMA,
        ],
    )
    def k(w_hbm, out_hbm, sem):
        wid = lax.axis_index("s") * nc + lax.axis_index("c")
        base = wid * rows_per_w
        src = w_hbm.at[pl.ds(base, rows_per_w)]
        cps = [
            pltpu.async_copy(
                src,
                out_hbm.at[pl.ds(b * seq_len + base, rows_per_w)],
                sem,
            )
            for b in range(bsz)
        ]
        for cp in cps:
            cp.wait()

    out = k(weights)
    return out.reshape(bsz, seq_len, embed_dim)


# speedup vs baseline: 22.0677x; 22.0677x over previous
import functools

import jax
import jax.numpy as jnp
from jax import lax
from jax.experimental import pallas as pl
from jax.experimental.pallas import tpu as pltpu
from jax.experimental.pallas import tpu_sc as plsc


def _sc_copy(weights, n_b, seq_len, embed_dim):
    info = plsc.get_sparse_core_info()
    nc, ns = info.num_cores, info.num_subcores
    nw = nc * ns
    rows_per_w = seq_len // nw
    chunk = 32
    n_chunks = rows_per_w // chunk

    mesh = plsc.VectorSubcoreMesh(core_axis_name="c", subcore_axis_name="s")

    @functools.partial(
        pl.kernel,
        mesh=mesh,
        out_type=jax.ShapeDtypeStruct((n_b * seq_len, embed_dim), jnp.float32),
        scratch_types=[
            pltpu.VMEM((chunk, embed_dim), jnp.float32),
            pltpu.VMEM((chunk, embed_dim), jnp.float32),
            pltpu.SemaphoreType.DMA,
            pltpu.SemaphoreType.DMA,
        ],
    )
    def k(w_hbm, out_hbm, buf0, buf1, sem_in, sem_out):
        wid = lax.axis_index("s") * nc + lax.axis_index("c")
        base = wid * rows_per_w
        bufs = (buf0, buf1)
        in_cp = {}
        out_cp = {}
        in_cp[0] = pltpu.async_copy(w_hbm.at[pl.ds(base, chunk)], bufs[0], sem_in)
        for i in range(n_chunks):
            if i + 1 < n_chunks:
                if i >= 1:
                    for cp in out_cp[i - 1]:
                        cp.wait()
                in_cp[i + 1] = pltpu.async_copy(
                    w_hbm.at[pl.ds(base + (i + 1) * chunk, chunk)],
                    bufs[(i + 1) % 2],
                    sem_in,
                )
            in_cp[i].wait()
            start = base + i * chunk
            out_cp[i] = [
                pltpu.async_copy(
                    bufs[i % 2],
                    out_hbm.at[pl.ds(b * seq_len + start, chunk)],
                    sem_out,
                )
                for b in range(n_b)
            ]
        for i in range(max(0, n_chunks - 2), n_chunks):
            for cp in out_cp[i]:
                cp.wait()

    return k(weights)


def _tc_copy(weights, n_b, seq_len, embed_dim):
    block_rows = 256

    def body(w_ref, o_ref):
        o_ref[...] = jnp.broadcast_to(
            w_ref[...][None, :, :], (n_b, block_rows, embed_dim)
        )

    return pl.pallas_call(
        body,
        grid=(seq_len // block_rows,),
        in_specs=[pl.BlockSpec((block_rows, embed_dim), lambda i: (i, 0))],
        out_specs=pl.BlockSpec(
            (n_b, block_rows, embed_dim), lambda i: (0, i, 0)
        ),
        out_shape=jax.ShapeDtypeStruct((n_b, seq_len, embed_dim), jnp.float32),
    )(weights)


def kernel(_input, weights):
    bsz, seq_len = _input.shape
    embed_dim = weights.shape[1]
    sc_b = 2
    tc_b = bsz - sc_b

    sc_out = _sc_copy(weights, sc_b, seq_len, embed_dim)
    tc_out = _tc_copy(weights, tc_b, seq_len, embed_dim)
    return jnp.concatenate(
        [tc_out, sc_out.reshape(sc_b, seq_len, embed_dim)], axis=0
    )


# speedup vs baseline: 45.1416x; 2.0456x over previous
import functools

import jax
import jax.numpy as jnp
from jax import lax
from jax.experimental import pallas as pl
from jax.experimental.pallas import tpu as pltpu
from jax.experimental.pallas import tpu_sc as plsc


def kernel(_input, weights):
    bsz, seq_len = _input.shape
    embed_dim = weights.shape[1]

    info = plsc.get_sparse_core_info()
    nc, ns = info.num_cores, info.num_subcores
    nw = nc * ns
    rows_per_w = seq_len // nw
    chunk = 32
    n_chunks = rows_per_w // chunk

    mesh = plsc.VectorSubcoreMesh(core_axis_name="c", subcore_axis_name="s")

    @functools.partial(
        pl.kernel,
        mesh=mesh,
        out_type=jax.ShapeDtypeStruct((bsz * seq_len, embed_dim), jnp.float32),
        scratch_types=[
            pltpu.VMEM((chunk, embed_dim), jnp.float32),
            pltpu.VMEM((chunk, embed_dim), jnp.float32),
            pltpu.VMEM((chunk, embed_dim), jnp.float32),
            pltpu.SemaphoreType.DMA,
            pltpu.SemaphoreType.DMA,
        ],
    )
    def k(w_hbm, out_hbm, buf0, buf1, buf2, sem_in, sem_out):
        wid = lax.axis_index("s") * nc + lax.axis_index("c")
        base = wid * rows_per_w
        bufs = (buf0, buf1, buf2)
        nbuf = len(bufs)
        in_cp = {}
        out_cp = {}
        in_cp[0] = pltpu.async_copy(w_hbm.at[pl.ds(base, chunk)], bufs[0], sem_in)
        for i in range(n_chunks):
            if i + 1 < n_chunks:
                if i + 1 >= nbuf:
                    for cp in out_cp[i + 1 - nbuf]:
                        cp.wait()
                in_cp[i + 1] = pltpu.async_copy(
                    w_hbm.at[pl.ds(base + (i + 1) * chunk, chunk)],
                    bufs[(i + 1) % nbuf],
                    sem_in,
                )
            in_cp[i].wait()
            start = base + i * chunk
            out_cp[i] = [
                pltpu.async_copy(
                    bufs[i % nbuf],
                    out_hbm.at[pl.ds(b * seq_len + start, chunk)],
                    sem_out,
                )
                for b in range(bsz)
            ]
        for i in range(max(0, n_chunks - nbuf + 1), n_chunks):
            for cp in out_cp[i]:
                cp.wait()

    out = k(weights)
    return out.reshape(bsz, seq_len, embed_dim)


# speedup vs baseline: 45.4178x; 1.0061x over previous
import functools

import jax
import jax.numpy as jnp
from jax import lax
from jax.experimental import pallas as pl
from jax.experimental.pallas import tpu as pltpu
from jax.experimental.pallas import tpu_sc as plsc


def kernel(_input, weights):
    bsz, seq_len = _input.shape
    embed_dim = weights.shape[1]

    info = plsc.get_sparse_core_info()
    nc, ns = info.num_cores, info.num_subcores
    nw = nc * ns
    rows_per_w = seq_len // nw
    chunk = 32
    n_chunks = rows_per_w // chunk

    mesh = plsc.VectorSubcoreMesh(core_axis_name="c", subcore_axis_name="s")

    @functools.partial(
        pl.kernel,
        mesh=mesh,
        out_type=jax.ShapeDtypeStruct((bsz * seq_len, embed_dim), jnp.float32),
        scratch_types=[
            pltpu.VMEM((chunk, embed_dim), jnp.float32),
            pltpu.VMEM((chunk, embed_dim), jnp.float32),
            pltpu.VMEM((chunk, embed_dim), jnp.float32),
            pltpu.SemaphoreType.DMA,
            pltpu.SemaphoreType.DMA,
        ],
    )
    def k(w_hbm, out_hbm, buf0, buf1, buf2, sem_in, sem_out):
        wid = lax.axis_index("s") * nc + lax.axis_index("c")
        base = wid * rows_per_w
        bufs = (buf0, buf1, buf2)
        nbuf = len(bufs)
        in_cp = {}
        out_cp = {}
        in_cp[0] = pltpu.async_copy(w_hbm.at[pl.ds(base, chunk)], bufs[0], sem_in)
        for i in range(n_chunks):
            if i + 1 < n_chunks:
                if i + 1 >= nbuf:
                    for cp in out_cp[i + 1 - nbuf]:
                        cp.wait()
                in_cp[i + 1] = pltpu.async_copy(
                    w_hbm.at[pl.ds(base + (i + 1) * chunk, chunk)],
                    bufs[(i + 1) % nbuf],
                    sem_in,
                )
            in_cp[i].wait()
            start = base + i * chunk
            out_cp[i] = [
                pltpu.async_copy(
                    bufs[i % nbuf],
                    out_hbm.at[pl.ds(b * seq_len + start, chunk)],
                    sem_out,
                )
                for b in range(bsz)
            ]
        for i in range(max(0, n_chunks - nbuf), n_chunks):
            for cp in out_cp[i]:
                cp.wait()

    out = k(weights)
    return out.reshape(bsz, seq_len, embed_dim)
